# Initial kernel scaffold; baseline (speedup 1.0000x reference)
import jax
import jax.numpy as jnp
from jax.experimental import pallas as pl


def kernel(features, edge_index, W_conv, b_conv, w_ih0, w_hh0, b_ih0, b_hh0, w_ih1, w_hh1, b_ih1, b_hh1):
    raise NotImplementedError("write your pallas kernel here")


# speedup vs baseline: 1.6026x; 1.6026x over previous
import functools

import jax
import jax.numpy as jnp
from jax import lax
from jax.experimental import pallas as pl
from jax.experimental.pallas import tpu as pltpu
from jax.experimental.pallas import tpu_sc as plsc

_NC = 2
_NS = 16


_DEG_K = 80
_DEG_ZR = 128
_DEG_STRIDE = 624


def _degree_body(sd_hbm, out_hbm, didx, ones_rows, zbuf, acc, sem):
    c = lax.axis_index("c")
    s = lax.axis_index("s")
    E = sd_hbm.shape[0] // 2
    K = _DEG_K
    epw = E // _NS
    nch = epw // K
    one = jnp.ones((16,), jnp.float32)
    zv = jnp.zeros((16,), jnp.float32)

    def fill(i, _):
        zbuf[i // 8, pl.ds((i % 8) * 16, 16)] = zv
        ones_rows[i // 8, pl.ds((i % 8) * 16, 16)] = one
        return 0

    lax.fori_loop(0, _DEG_ZR * 8, fill, 0)
    rbase = s * _DEG_STRIDE
    for z in range(5):
        pltpu.sync_copy(zbuf, acc.at[pl.ds(rbase + z * _DEG_ZR, _DEG_ZR)])
    plsc.subcore_barrier()

    def chunk(ch, _):
        ebase = c * E + s * epw + ch * K
        pltpu.sync_copy(sd_hbm.at[pl.ds(ebase, K)], didx)
        pltpu.sync_copy(ones_rows.at[pl.ds(0, K)], acc.at[didx], add=True)
        return 0

    lax.fori_loop(0, nch, chunk, 0)
    plsc.subcore_barrier()
    for z in range(5):
        rb = rbase + z * _DEG_ZR
        pltpu.sync_copy(acc.at[pl.ds(rb, _DEG_ZR)],
                        out_hbm.at[c, pl.ds(rb, _DEG_ZR)])


def _degrees(src, dst, n):
    d = 128
    sd = jnp.concatenate([src, dst])
    return pl.kernel(
        _degree_body,
        out_type=jax.ShapeDtypeStruct((2, n, d), jnp.float32),
        mesh=plsc.VectorSubcoreMesh(core_axis_name="c", subcore_axis_name="s",
                                    num_cores=_NC, num_subcores=_NS),
        scratch_types=[
            pltpu.VMEM((_DEG_K,), jnp.int32),
            pltpu.VMEM((_DEG_ZR, d), jnp.float32),
            pltpu.VMEM((_DEG_ZR, d), jnp.float32),
            pltpu.VMEM_SHARED((n, d), jnp.float32),
            pltpu.SemaphoreType.DMA,
        ],
    )(sd)


_SEG_K = 80
_SEG_ZR = 128
_SEG_STRIDE = 624


def _segsum_body(h_hbm, src_hbm, dst_hbm, out_hbm, sidx, didx, rows, zbuf,
                 acc, sem):
    c = lax.axis_index("c")
    s = lax.axis_index("s")
    E = src_hbm.shape[0]
    N = acc.shape[0]
    K = _SEG_K
    epw = E // _NS
    nch = epw // K
    zv = jnp.zeros((16,), jnp.float32)

    def zb(i, _):
        zbuf[i // 8, pl.ds((i % 8) * 16, 16)] = zv
        return 0

    lax.fori_loop(0, _SEG_ZR * 8, zb, 0)
    rbase = s * _SEG_STRIDE

    for t_local in range(2):
        t = c * 2 + t_local
        toff = t * N
        for z in range(5):
            pltpu.sync_copy(zbuf, acc.at[pl.ds(rbase + z * _SEG_ZR, _SEG_ZR)])
        plsc.subcore_barrier()

        def chunk(ch, _):
            ebase = s * epw + ch * K
            pltpu.sync_copy(src_hbm.at[pl.ds(ebase, K)], sidx)
            pltpu.sync_copy(dst_hbm.at[pl.ds(ebase, K)], didx)
            tvec = jnp.full((16,), toff, jnp.int32)
            for j in range(K // 16):
                sidx[pl.ds(j * 16, 16)] = sidx[pl.ds(j * 16, 16)] + tvec
            pltpu.async_copy(h_hbm.at[sidx], rows, sem).wait()
            pltpu.sync_copy(rows, acc.at[didx], add=True)
            return 0

        lax.fori_loop(0, nch, chunk, 0)
        plsc.subcore_barrier()
        for z in range(5):
            rb = rbase + z * _SEG_ZR
            pltpu.sync_copy(acc.at[pl.ds(rb, _SEG_ZR)],
                            out_hbm.at[pl.ds(toff + rb, _SEG_ZR)])
        plsc.subcore_barrier()


def _segsum(h2, src, dst, n):
    tn, d = h2.shape
    return pl.kernel(
        _segsum_body,
        out_type=jax.ShapeDtypeStruct((tn, d), jnp.float32),
        mesh=plsc.VectorSubcoreMesh(core_axis_name="c", subcore_axis_name="s",
                                    num_cores=_NC, num_subcores=_NS),
        scratch_types=[
            pltpu.VMEM((_SEG_K,), jnp.int32),
            pltpu.VMEM((_SEG_K,), jnp.int32),
            pltpu.VMEM((_SEG_K, d), jnp.float32),
            pltpu.VMEM((_SEG_ZR, d), jnp.float32),
            pltpu.VMEM_SHARED((n, d), jnp.float32),
            pltpu.SemaphoreType.DMA,
        ],
    )(h2, src, dst)


_BLK = 2000


def _c0_body(x_ref, degs_ref, h_ref, nin0_ref, nout1_ref, nin1_ref):
    T = x_ref.shape[0]
    deg_out = degs_ref[0, :, 0:1]
    deg_in = degs_ref[1, :, 0:1]
    nout0 = lax.rsqrt(jnp.maximum(deg_out, 1.0))
    nin0_ref[...] = lax.rsqrt(jnp.maximum(deg_in, 1.0))
    nout1_ref[...] = lax.rsqrt(jnp.maximum(deg_out + 1.0, 1.0))
    nin1_ref[...] = lax.rsqrt(jnp.maximum(deg_in + 1.0, 1.0))
    for t in range(T):
        h_ref[t] = x_ref[t] * nout0


def _c0(features, degs):
    t, n, d = features.shape
    nb = n // _BLK
    return pl.pallas_call(
        _c0_body,
        grid=(nb,),
        in_specs=[
            pl.BlockSpec((t, _BLK, d), lambda i: (0, i, 0)),
            pl.BlockSpec((2, _BLK, 128), lambda i: (0, i, 0)),
        ],
        out_specs=[
            pl.BlockSpec((t, _BLK, d), lambda i: (0, i, 0)),
            pl.BlockSpec((_BLK, 1), lambda i: (i, 0)),
            pl.BlockSpec((_BLK, 1), lambda i: (i, 0)),
            pl.BlockSpec((_BLK, 1), lambda i: (i, 0)),
        ],
        out_shape=[
            jax.ShapeDtypeStruct((t, n, d), jnp.float32),
            jax.ShapeDtypeStruct((n, 1), jnp.float32),
            jax.ShapeDtypeStruct((n, 1), jnp.float32),
            jax.ShapeDtypeStruct((n, 1), jnp.float32),
        ],
    )(features, degs)


def _c1_body(agg_ref, nin_ref, nout_ref, w_ref, b_ref, h_ref):
    a = agg_ref[0] * nin_ref[...]
    x1 = jnp.dot(a, w_ref[...], preferred_element_type=jnp.float32) + b_ref[...]
    h_ref[0] = x1 * nout_ref[...]


def _c1(agg0, nin0, nout1, w, b):
    t, n, d = agg0.shape
    h = w.shape[1]
    nb = n // _BLK
    return pl.pallas_call(
        _c1_body,
        grid=(t, nb),
        in_specs=[
            pl.BlockSpec((1, _BLK, d), lambda ti, i: (ti, i, 0)),
            pl.BlockSpec((_BLK, 1), lambda ti, i: (i, 0)),
            pl.BlockSpec((_BLK, 1), lambda ti, i: (i, 0)),
            pl.BlockSpec((d, h), lambda ti, i: (0, 0)),
            pl.BlockSpec((1, h), lambda ti, i: (0, 0)),
        ],
        out_specs=pl.BlockSpec((1, _BLK, h), lambda ti, i: (ti, i, 0)),
        out_shape=jax.ShapeDtypeStruct((t, n, h), jnp.float32),
    )(agg0, nin0, nout1, w, b)


def _c2_body(agg_ref, h1_ref, nin_ref, w_ref, b_ref, out_ref):
    T = agg_ref.shape[0]
    nin = nin_ref[...]
    w = w_ref[...]
    b = b_ref[...]
    for t in range(T):
        a = (agg_ref[t] + h1_ref[t]) * nin
        out_ref[:, t, :] = jnp.dot(
            a, w, preferred_element_type=jnp.float32) + b


def _c2(agg1, h1, nin1, w, b):
    t, n, d = agg1.shape
    h = w.shape[1]
    nb = n // _BLK
    return pl.pallas_call(
        _c2_body,
        grid=(nb,),
        in_specs=[
            pl.BlockSpec((t, _BLK, d), lambda i: (0, i, 0)),
            pl.BlockSpec((t, _BLK, d), lambda i: (0, i, 0)),
            pl.BlockSpec((_BLK, 1), lambda i: (i, 0)),
            pl.BlockSpec((d, h), lambda i: (0, 0)),
            pl.BlockSpec((1, h), lambda i: (0, 0)),
        ],
        out_specs=pl.BlockSpec((_BLK, t, h), lambda i: (i, 0, 0)),
        out_shape=jax.ShapeDtypeStruct((n, t, h), jnp.float32),
    )(agg1, h1, nin1, w, b)


_CHUNK = 500
_DN = (((1,), (1,)), ((), ()))
_PREC = lax.Precision.HIGHEST


def _gru_body(x_ref, wi0, wh0, wi1, wh1, bi0, bh0, bi1, bh1,
              out_ref, gi_ref, hs_ref):
    chunk, T, D = x_ref.shape
    H = D
    c = pl.program_id(0)
    nc = pl.num_programs(0)

    @pl.when(c == 0)
    def _():
        hs_ref[...] = jnp.zeros_like(hs_ref)

    x = x_ref[...].reshape(chunk * T, D)
    gi_ref[...] = lax.dot_general(
        x, wi0[...], _DN, preferred_element_type=jnp.float32,
        precision=_PREC) + bi0[...]

    def step2(i, _):
        base = pl.multiple_of(8 * i, 8)
        g8 = gi_ref[pl.ds(base, 8), :]
        for k in range(2):
            g0 = g8[4 * k:4 * k + 4, :]
            h0 = hs_ref[0:4, :]
            h1 = hs_ref[4:8, :]
            gh0 = lax.dot_general(
                h0, wh0[...], _DN, preferred_element_type=jnp.float32,
                precision=_PREC) + bh0[...]
            r0 = jax.nn.sigmoid(g0[:, 0:H] + gh0[:, 0:H])
            z0 = jax.nn.sigmoid(g0[:, H:2 * H] + gh0[:, H:2 * H])
            n0 = jnp.tanh(g0[:, 2 * H:3 * H] + r0 * gh0[:, 2 * H:3 * H])
            h0n = (1.0 - z0) * n0 + z0 * h0
            gi1 = lax.dot_general(
                h0n, wi1[...], _DN, preferred_element_type=jnp.float32,
                precision=_PREC) + bi1[...]
            gh1 = lax.dot_general(
                h1, wh1[...], _DN, preferred_element_type=jnp.float32,
                precision=_PREC) + bh1[...]
            r1 = jax.nn.sigmoid(gi1[:, 0:H] + gh1[:, 0:H])
            z1 = jax.nn.sigmoid(gi1[:, H:2 * H] + gh1[:, H:2 * H])
            n1 = jnp.tanh(gi1[:, 2 * H:3 * H] + r1 * gh1[:, 2 * H:3 * H])
            h1n = (1.0 - z1) * n1 + z1 * h1
            hs_ref[0:4, :] = h0n
            hs_ref[4:8, :] = h1n
        return 0

    lax.fori_loop(0, chunk // 2, step2, 0)

    @pl.when(c == nc - 1)
    def _():
        out_ref[...] = jnp.maximum(hs_ref[4:8, :], 0.0)


def _gru(x, wi0, wh0, wi1, wh1, bi0, bh0, bi1, bh1):
    n, t, d = x.shape
    h3 = wi0.shape[0]
    nc = n // _CHUNK
    wspec = pl.BlockSpec((h3, d), lambda c: (0, 0))
    bspec = pl.BlockSpec((1, h3), lambda c: (0, 0))
    return pl.pallas_call(
        _gru_body,
        grid=(nc,),
        in_specs=[
            pl.BlockSpec((_CHUNK, t, d), lambda c: (c, 0, 0)),
            wspec, wspec, wspec, wspec, bspec, bspec, bspec, bspec,
        ],
        out_specs=pl.BlockSpec((t, d), lambda c: (0, 0)),
        out_shape=jax.ShapeDtypeStruct((t, d), jnp.float32),
        scratch_shapes=[
            pltpu.VMEM((_CHUNK * t, h3), jnp.float32),
            pltpu.VMEM((8, d), jnp.float32),
        ],
    )(x, wi0, wh0, wi1, wh1, bi0, bh0, bi1, bh1)


def kernel(features, edge_index, W_conv, b_conv, w_ih0, w_hh0, b_ih0, b_hh0,
           w_ih1, w_hh1, b_ih1, b_hh1):
    t, n, d = features.shape
    h = W_conv.shape[1]
    src = edge_index[0]
    dst = edge_index[1]
    degs = _degrees(src, dst, n)
    h0, nin0, nout1, nin1 = _c0(features, degs)
    agg0 = _segsum(h0.reshape(t * n, d), src, dst, n)
    h1 = _c1(agg0.reshape(t, n, d), nin0, nout1, W_conv, b_conv.reshape(1, h))
    agg1 = _segsum(h1.reshape(t * n, h), src, dst, n)
    x2 = _c2(agg1.reshape(t, n, h), h1, nin1, W_conv, b_conv.reshape(1, h))
    out = _gru(x2, w_ih0, w_hh0, w_ih1, w_hh1,
               b_ih0.reshape(1, -1), b_hh0.reshape(1, -1),
               b_ih1.reshape(1, -1), b_hh1.reshape(1, -1))
    return out[:, None, :]


# speedup vs baseline: 1.6414x; 1.0242x over previous
import functools

import jax
import jax.numpy as jnp
from jax import lax
from jax.experimental import pallas as pl
from jax.experimental.pallas import tpu as pltpu
from jax.experimental.pallas import tpu_sc as plsc

_NC = 2
_NS = 16


_DEG_K = 80
_DEG_ZR = 128
_DEG_STRIDE = 624


def _degree_body(sd_hbm, out_hbm, didx, ones_rows, zbuf, acc, sem):
    c = lax.axis_index("c")
    s = lax.axis_index("s")
    E = sd_hbm.shape[0] // 2
    K = _DEG_K
    epw = E // _NS
    nch = epw // K
    one = jnp.ones((16,), jnp.float32)
    zv = jnp.zeros((16,), jnp.float32)

    def fill(i, _):
        zbuf[i // 8, pl.ds((i % 8) * 16, 16)] = zv
        ones_rows[i // 8, pl.ds((i % 8) * 16, 16)] = one
        return 0

    lax.fori_loop(0, _DEG_ZR * 8, fill, 0)
    rbase = s * _DEG_STRIDE
    for z in range(5):
        pltpu.sync_copy(zbuf, acc.at[pl.ds(rbase + z * _DEG_ZR, _DEG_ZR)])
    plsc.subcore_barrier()

    def chunk(ch, _):
        ebase = c * E + s * epw + ch * K
        pltpu.sync_copy(sd_hbm.at[pl.ds(ebase, K)], didx)
        pltpu.sync_copy(ones_rows.at[pl.ds(0, K)], acc.at[didx], add=True)
        return 0

    lax.fori_loop(0, nch, chunk, 0)
    plsc.subcore_barrier()
    for z in range(5):
        rb = rbase + z * _DEG_ZR
        pltpu.sync_copy(acc.at[pl.ds(rb, _DEG_ZR)],
                        out_hbm.at[c, pl.ds(rb, _DEG_ZR)])


def _degrees(src, dst, n):
    d = 128
    sd = jnp.concatenate([src, dst])
    return pl.kernel(
        _degree_body,
        out_type=jax.ShapeDtypeStruct((2, n, d), jnp.float32),
        mesh=plsc.VectorSubcoreMesh(core_axis_name="c", subcore_axis_name="s",
                                    num_cores=_NC, num_subcores=_NS),
        scratch_types=[
            pltpu.VMEM((_DEG_K,), jnp.int32),
            pltpu.VMEM((_DEG_ZR, d), jnp.float32),
            pltpu.VMEM((_DEG_ZR, d), jnp.float32),
            pltpu.VMEM_SHARED((n, d), jnp.float32),
            pltpu.SemaphoreType.DMA,
        ],
    )(sd)


_SEG_K = 80
_SEG_ZR = 128
_SEG_STRIDE = 624


def _segsum_body(h_hbm, src_hbm, dst_hbm, out_hbm, sidx, didx, rows, zbuf,
                 acc, sem):
    c = lax.axis_index("c")
    s = lax.axis_index("s")
    E = src_hbm.shape[0]
    N = acc.shape[0]
    K = _SEG_K
    epw = E // _NS
    nch = epw // K
    zv = jnp.zeros((16,), jnp.float32)

    def zb(i, _):
        zbuf[i // 8, pl.ds((i % 8) * 16, 16)] = zv
        return 0

    lax.fori_loop(0, _SEG_ZR * 8, zb, 0)
    rbase = s * _SEG_STRIDE

    for t_local in range(2):
        t = c * 2 + t_local
        toff = t * N
        for z in range(5):
            pltpu.sync_copy(zbuf, acc.at[pl.ds(rbase + z * _SEG_ZR, _SEG_ZR)])
        plsc.subcore_barrier()

        def chunk(ch, _):
            ebase = s * epw + ch * K
            pltpu.sync_copy(src_hbm.at[pl.ds(ebase, K)], sidx)
            pltpu.sync_copy(dst_hbm.at[pl.ds(ebase, K)], didx)
            tvec = jnp.full((16,), toff, jnp.int32)
            for j in range(K // 16):
                sidx[pl.ds(j * 16, 16)] = sidx[pl.ds(j * 16, 16)] + tvec
            pltpu.async_copy(h_hbm.at[sidx], rows, sem).wait()
            pltpu.sync_copy(rows, acc.at[didx], add=True)
            return 0

        lax.fori_loop(0, nch, chunk, 0)
        plsc.subcore_barrier()
        for z in range(5):
            rb = rbase + z * _SEG_ZR
            pltpu.sync_copy(acc.at[pl.ds(rb, _SEG_ZR)],
                            out_hbm.at[pl.ds(toff + rb, _SEG_ZR)])
        plsc.subcore_barrier()


def _segsum(h2, src, dst, n):
    tn, d = h2.shape
    return pl.kernel(
        _segsum_body,
        out_type=jax.ShapeDtypeStruct((tn, d), jnp.float32),
        mesh=plsc.VectorSubcoreMesh(core_axis_name="c", subcore_axis_name="s",
                                    num_cores=_NC, num_subcores=_NS),
        scratch_types=[
            pltpu.VMEM((_SEG_K,), jnp.int32),
            pltpu.VMEM((_SEG_K,), jnp.int32),
            pltpu.VMEM((_SEG_K, d), jnp.float32),
            pltpu.VMEM((_SEG_ZR, d), jnp.float32),
            pltpu.VMEM_SHARED((n, d), jnp.float32),
            pltpu.SemaphoreType.DMA,
        ],
    )(h2, src, dst)


_BLK = 2000


def _c0_body(x_ref, degs_ref, h_ref, nin0_ref, nout1_ref, nin1_ref):
    T = x_ref.shape[0]
    deg_out = degs_ref[0, :, 0:1]
    deg_in = degs_ref[1, :, 0:1]
    nout0 = lax.rsqrt(jnp.maximum(deg_out, 1.0))
    nin0_ref[...] = lax.rsqrt(jnp.maximum(deg_in, 1.0))
    nout1_ref[...] = lax.rsqrt(jnp.maximum(deg_out + 1.0, 1.0))
    nin1_ref[...] = lax.rsqrt(jnp.maximum(deg_in + 1.0, 1.0))
    for t in range(T):
        h_ref[t] = x_ref[t] * nout0


def _c0(features, degs):
    t, n, d = features.shape
    nb = n // _BLK
    return pl.pallas_call(
        _c0_body,
        grid=(nb,),
        in_specs=[
            pl.BlockSpec((t, _BLK, d), lambda i: (0, i, 0)),
            pl.BlockSpec((2, _BLK, 128), lambda i: (0, i, 0)),
        ],
        out_specs=[
            pl.BlockSpec((t, _BLK, d), lambda i: (0, i, 0)),
            pl.BlockSpec((_BLK, 1), lambda i: (i, 0)),
            pl.BlockSpec((_BLK, 1), lambda i: (i, 0)),
            pl.BlockSpec((_BLK, 1), lambda i: (i, 0)),
        ],
        out_shape=[
            jax.ShapeDtypeStruct((t, n, d), jnp.float32),
            jax.ShapeDtypeStruct((n, 1), jnp.float32),
            jax.ShapeDtypeStruct((n, 1), jnp.float32),
            jax.ShapeDtypeStruct((n, 1), jnp.float32),
        ],
    )(features, degs)


def _c1_body(agg_ref, nin_ref, nout_ref, w_ref, b_ref, h_ref):
    a = agg_ref[0] * nin_ref[...]
    x1 = jnp.dot(a, w_ref[...], preferred_element_type=jnp.float32) + b_ref[...]
    h_ref[0] = x1 * nout_ref[...]


def _c1(agg0, nin0, nout1, w, b):
    t, n, d = agg0.shape
    h = w.shape[1]
    nb = n // _BLK
    return pl.pallas_call(
        _c1_body,
        grid=(t, nb),
        in_specs=[
            pl.BlockSpec((1, _BLK, d), lambda ti, i: (ti, i, 0)),
            pl.BlockSpec((_BLK, 1), lambda ti, i: (i, 0)),
            pl.BlockSpec((_BLK, 1), lambda ti, i: (i, 0)),
            pl.BlockSpec((d, h), lambda ti, i: (0, 0)),
            pl.BlockSpec((1, h), lambda ti, i: (0, 0)),
        ],
        out_specs=pl.BlockSpec((1, _BLK, h), lambda ti, i: (ti, i, 0)),
        out_shape=jax.ShapeDtypeStruct((t, n, h), jnp.float32),
    )(agg0, nin0, nout1, w, b)


def _c2_body(agg_ref, h1_ref, nin_ref, w_ref, b_ref, out_ref):
    T = agg_ref.shape[0]
    nin = nin_ref[...]
    w = w_ref[...]
    b = b_ref[...]
    for t in range(T):
        a = (agg_ref[t] + h1_ref[t]) * nin
        out_ref[:, t, :] = jnp.dot(
            a, w, preferred_element_type=jnp.float32) + b


def _c2(agg1, h1, nin1, w, b):
    t, n, d = agg1.shape
    h = w.shape[1]
    nb = n // _BLK
    return pl.pallas_call(
        _c2_body,
        grid=(nb,),
        in_specs=[
            pl.BlockSpec((t, _BLK, d), lambda i: (0, i, 0)),
            pl.BlockSpec((t, _BLK, d), lambda i: (0, i, 0)),
            pl.BlockSpec((_BLK, 1), lambda i: (i, 0)),
            pl.BlockSpec((d, h), lambda i: (0, 0)),
            pl.BlockSpec((1, h), lambda i: (0, 0)),
        ],
        out_specs=pl.BlockSpec((_BLK, t, h), lambda i: (i, 0, 0)),
        out_shape=jax.ShapeDtypeStruct((n, t, h), jnp.float32),
    )(agg1, h1, nin1, w, b)


_CHUNK = 500
_DN = (((1,), (1,)), ((), ()))
_PREC = lax.Precision.HIGHEST


def _gru_body(x_ref, wi0, whc, wi1, bi0, bhc, bi1,
              out_ref, gi_ref, hs_ref):
    chunk, T, D = x_ref.shape
    H = D
    c = pl.program_id(0)
    nc = pl.num_programs(0)

    @pl.when(c == 0)
    def _():
        hs_ref[...] = jnp.zeros_like(hs_ref)

    x = x_ref[...].reshape(chunk * T, D)
    gi_ref[...] = lax.dot_general(
        x, wi0[...], _DN, preferred_element_type=jnp.float32,
        precision=_PREC) + bi0[...]

    def step2(i, _):
        base = pl.multiple_of(8 * i, 8)
        g8 = gi_ref[pl.ds(base, 8), :]
        for k in range(2):
            g0 = g8[4 * k:4 * k + 4, :]
            hs = hs_ref[...]
            h0 = hs[:, 0:H]
            h1 = hs[:, H:2 * H]
            gh = jnp.dot(hs, whc[...], preferred_element_type=jnp.float32,
                         precision=_PREC) + bhc[...]
            r0 = jax.nn.sigmoid(g0[:, 0:H] + gh[:, 0:H])
            z0 = jax.nn.sigmoid(g0[:, H:2 * H] + gh[:, H:2 * H])
            n0 = jnp.tanh(g0[:, 2 * H:3 * H] + r0 * gh[:, 2 * H:3 * H])
            h0n = n0 + z0 * (h0 - n0)
            gi1 = lax.dot_general(
                h0n, wi1[...], _DN, preferred_element_type=jnp.float32,
                precision=_PREC) + bi1[...]
            r1 = jax.nn.sigmoid(gi1[:, 0:H] + gh[:, 3 * H:4 * H])
            z1 = jax.nn.sigmoid(gi1[:, H:2 * H] + gh[:, 4 * H:5 * H])
            n1 = jnp.tanh(gi1[:, 2 * H:3 * H] + r1 * gh[:, 5 * H:6 * H])
            h1n = n1 + z1 * (h1 - n1)
            hs_ref[:, 0:H] = h0n
            hs_ref[:, H:2 * H] = h1n
        return 0

    lax.fori_loop(0, chunk // 2, step2, 0)

    @pl.when(c == nc - 1)
    def _():
        out_ref[...] = jnp.maximum(hs_ref[:, H:2 * H], 0.0)


def _gru(x, wi0, whc, wi1, bi0, bhc, bi1):
    n, t, d = x.shape
    h3 = wi0.shape[0]
    nc = n // _CHUNK
    return pl.pallas_call(
        _gru_body,
        grid=(nc,),
        in_specs=[
            pl.BlockSpec((_CHUNK, t, d), lambda c: (c, 0, 0)),
            pl.BlockSpec(wi0.shape, lambda c: (0, 0)),
            pl.BlockSpec(whc.shape, lambda c: (0, 0)),
            pl.BlockSpec(wi1.shape, lambda c: (0, 0)),
            pl.BlockSpec(bi0.shape, lambda c: (0, 0)),
            pl.BlockSpec(bhc.shape, lambda c: (0, 0)),
            pl.BlockSpec(bi1.shape, lambda c: (0, 0)),
        ],
        out_specs=pl.BlockSpec((t, d), lambda c: (0, 0)),
        out_shape=jax.ShapeDtypeStruct((t, d), jnp.float32),
        scratch_shapes=[
            pltpu.VMEM((_CHUNK * t, h3), jnp.float32),
            pltpu.VMEM((t, 2 * d), jnp.float32),
        ],
    )(x, wi0, whc, wi1, bi0, bhc, bi1)


def kernel(features, edge_index, W_conv, b_conv, w_ih0, w_hh0, b_ih0, b_hh0,
           w_ih1, w_hh1, b_ih1, b_hh1):
    t, n, d = features.shape
    h = W_conv.shape[1]
    src = edge_index[0]
    dst = edge_index[1]
    degs = _degrees(src, dst, n)
    h0, nin0, nout1, nin1 = _c0(features, degs)
    agg0 = _segsum(h0.reshape(t * n, d), src, dst, n)
    h1 = _c1(agg0.reshape(t, n, d), nin0, nout1, W_conv, b_conv.reshape(1, h))
    agg1 = _segsum(h1.reshape(t * n, h), src, dst, n)
    x2 = _c2(agg1.reshape(t, n, h), h1, nin1, W_conv, b_conv.reshape(1, h))
    whc = jnp.zeros((2 * h, 6 * h), jnp.float32)
    whc = whc.at[0:h, 0:3 * h].set(w_hh0.T)
    whc = whc.at[h:2 * h, 3 * h:6 * h].set(w_hh1.T)
    bhc = jnp.concatenate([b_hh0, b_hh1]).reshape(1, -1)
    out = _gru(x2, w_ih0, whc, w_ih1,
               b_ih0.reshape(1, -1), bhc, b_ih1.reshape(1, -1))
    return out[:, None, :]


# speedup vs baseline: 1.8217x; 1.1098x over previous
import functools

import jax
import jax.numpy as jnp
from jax import lax
from jax.experimental import pallas as pl
from jax.experimental.pallas import tpu as pltpu
from jax.experimental.pallas import tpu_sc as plsc

_NC = 2
_NS = 16


_DEG_K = 80
_DEG_ZR = 128
_DEG_STRIDE = 624


def _degree_body(sd_hbm, out_hbm, didx, ones_rows, zbuf, acc, sem):
    c = lax.axis_index("c")
    s = lax.axis_index("s")
    E = sd_hbm.shape[0] // 2
    K = _DEG_K
    epw = E // _NS
    nch = epw // K
    one = jnp.ones((16,), jnp.float32)
    zv = jnp.zeros((16,), jnp.float32)

    def fill(i, _):
        zbuf[i // 8, pl.ds((i % 8) * 16, 16)] = zv
        ones_rows[i // 8, pl.ds((i % 8) * 16, 16)] = one
        return 0

    lax.fori_loop(0, _DEG_ZR * 8, fill, 0)
    rbase = s * _DEG_STRIDE
    for z in range(5):
        pltpu.sync_copy(zbuf, acc.at[pl.ds(rbase + z * _DEG_ZR, _DEG_ZR)])
    plsc.subcore_barrier()

    def chunk(ch, _):
        ebase = c * E + s * epw + ch * K
        pltpu.sync_copy(sd_hbm.at[pl.ds(ebase, K)], didx)
        pltpu.sync_copy(ones_rows.at[pl.ds(0, K)], acc.at[didx], add=True)
        return 0

    lax.fori_loop(0, nch, chunk, 0)
    plsc.subcore_barrier()
    for z in range(5):
        rb = rbase + z * _DEG_ZR
        pltpu.sync_copy(acc.at[pl.ds(rb, _DEG_ZR)],
                        out_hbm.at[c, pl.ds(rb, _DEG_ZR)])


def _degrees(src, dst, n):
    d = 128
    sd = jnp.concatenate([src, dst])
    return pl.kernel(
        _degree_body,
        out_type=jax.ShapeDtypeStruct((2, n, d), jnp.float32),
        mesh=plsc.VectorSubcoreMesh(core_axis_name="c", subcore_axis_name="s",
                                    num_cores=_NC, num_subcores=_NS),
        scratch_types=[
            pltpu.VMEM((_DEG_K,), jnp.int32),
            pltpu.VMEM((_DEG_ZR, d), jnp.float32),
            pltpu.VMEM((_DEG_ZR, d), jnp.float32),
            pltpu.VMEM_SHARED((n, d), jnp.float32),
            pltpu.SemaphoreType.DMA,
        ],
    )(sd)


_SEG_K = 80
_SEG_ZR = 128
_SEG_STRIDE = 624


def _segsum_body(h_hbm, src_hbm, dst_hbm, out_hbm, sidx2, didx2, rows2,
                 zbuf, acc, sem0, sem1):
    c = lax.axis_index("c")
    s = lax.axis_index("s")
    E = src_hbm.shape[0]
    N = acc.shape[0]
    K = _SEG_K
    epw = E // _NS
    nch = epw // K
    sems = (sem0, sem1)
    zv = jnp.zeros((16,), jnp.float32)

    def zb(i, _):
        zbuf[i // 8, pl.ds((i % 8) * 16, 16)] = zv
        return 0

    lax.fori_loop(0, _SEG_ZR * 8, zb, 0)
    rbase = s * _SEG_STRIDE

    for t_local in range(2):
        toff = (c * 2 + t_local) * N
        tvec = jnp.full((16,), toff, jnp.int32)
        for z in range(5):
            pltpu.sync_copy(zbuf, acc.at[pl.ds(rbase + z * _SEG_ZR, _SEG_ZR)])
        plsc.subcore_barrier()

        def load_and_gather(ch, b):
            ebase = s * epw + ch * K
            pltpu.sync_copy(src_hbm.at[pl.ds(ebase, K)], sidx2.at[b])
            for j in range(K // 16):
                sidx2[b, pl.ds(j * 16, 16)] = \
                    sidx2[b, pl.ds(j * 16, 16)] + tvec
            pltpu.sync_copy(dst_hbm.at[pl.ds(ebase, K)], didx2.at[b])
            pltpu.async_copy(h_hbm.at[sidx2.at[b]], rows2.at[b], sems[b])

        def drain(b):
            pltpu.make_async_copy(
                h_hbm.at[sidx2.at[b]], rows2.at[b], sems[b]).wait()

        for b in range(2):
            load_and_gather(b, b)

        def pair(i, _):
            for b in range(2):
                ch = 2 * i + b
                drain(b)
                pltpu.sync_copy(rows2.at[b], acc.at[didx2.at[b]], add=True)
                load_and_gather(ch + 2, b)
            return 0

        lax.fori_loop(0, nch // 2 - 1, pair, 0)
        for b in range(2):
            drain(b)
            pltpu.sync_copy(rows2.at[b], acc.at[didx2.at[b]], add=True)
        plsc.subcore_barrier()
        for z in range(5):
            rb = rbase + z * _SEG_ZR
            pltpu.sync_copy(acc.at[pl.ds(rb, _SEG_ZR)],
                            out_hbm.at[pl.ds(toff + rb, _SEG_ZR)])
        plsc.subcore_barrier()


def _segsum(h2, src, dst, n):
    tn, d = h2.shape
    return pl.kernel(
        _segsum_body,
        out_type=jax.ShapeDtypeStruct((tn, d), jnp.float32),
        mesh=plsc.VectorSubcoreMesh(core_axis_name="c", subcore_axis_name="s",
                                    num_cores=_NC, num_subcores=_NS),
        scratch_types=[
            pltpu.VMEM((2, _SEG_K), jnp.int32),
            pltpu.VMEM((2, _SEG_K), jnp.int32),
            pltpu.VMEM((2, _SEG_K, d), jnp.float32),
            pltpu.VMEM((_SEG_ZR, d), jnp.float32),
            pltpu.VMEM_SHARED((n, d), jnp.float32),
            pltpu.SemaphoreType.DMA,
            pltpu.SemaphoreType.DMA,
        ],
    )(h2, src, dst)


_BLK = 2000


def _c0_body(x_ref, degs_ref, h_ref, nin0_ref, nout1_ref, nin1_ref):
    T = x_ref.shape[0]
    deg_out = degs_ref[0, :, 0:1]
    deg_in = degs_ref[1, :, 0:1]
    nout0 = lax.rsqrt(jnp.maximum(deg_out, 1.0))
    nin0_ref[...] = lax.rsqrt(jnp.maximum(deg_in, 1.0))
    nout1_ref[...] = lax.rsqrt(jnp.maximum(deg_out + 1.0, 1.0))
    nin1_ref[...] = lax.rsqrt(jnp.maximum(deg_in + 1.0, 1.0))
    for t in range(T):
        h_ref[t] = x_ref[t] * nout0


def _c0(features, degs):
    t, n, d = features.shape
    nb = n // _BLK
    return pl.pallas_call(
        _c0_body,
        grid=(nb,),
        in_specs=[
            pl.BlockSpec((t, _BLK, d), lambda i: (0, i, 0)),
            pl.BlockSpec((2, _BLK, 128), lambda i: (0, i, 0)),
        ],
        out_specs=[
            pl.BlockSpec((t, _BLK, d), lambda i: (0, i, 0)),
            pl.BlockSpec((_BLK, 1), lambda i: (i, 0)),
            pl.BlockSpec((_BLK, 1), lambda i: (i, 0)),
            pl.BlockSpec((_BLK, 1), lambda i: (i, 0)),
        ],
        out_shape=[
            jax.ShapeDtypeStruct((t, n, d), jnp.float32),
            jax.ShapeDtypeStruct((n, 1), jnp.float32),
            jax.ShapeDtypeStruct((n, 1), jnp.float32),
            jax.ShapeDtypeStruct((n, 1), jnp.float32),
        ],
    )(features, degs)


def _c1_body(agg_ref, nin_ref, nout_ref, w_ref, b_ref, h_ref):
    a = agg_ref[0] * nin_ref[...]
    x1 = jnp.dot(a, w_ref[...], preferred_element_type=jnp.float32) + b_ref[...]
    h_ref[0] = x1 * nout_ref[...]


def _c1(agg0, nin0, nout1, w, b):
    t, n, d = agg0.shape
    h = w.shape[1]
    nb = n // _BLK
    return pl.pallas_call(
        _c1_body,
        grid=(t, nb),
        in_specs=[
            pl.BlockSpec((1, _BLK, d), lambda ti, i: (ti, i, 0)),
            pl.BlockSpec((_BLK, 1), lambda ti, i: (i, 0)),
            pl.BlockSpec((_BLK, 1), lambda ti, i: (i, 0)),
            pl.BlockSpec((d, h), lambda ti, i: (0, 0)),
            pl.BlockSpec((1, h), lambda ti, i: (0, 0)),
        ],
        out_specs=pl.BlockSpec((1, _BLK, h), lambda ti, i: (ti, i, 0)),
        out_shape=jax.ShapeDtypeStruct((t, n, h), jnp.float32),
    )(agg0, nin0, nout1, w, b)


def _c2_body(agg_ref, h1_ref, nin_ref, w_ref, b_ref, out_ref):
    T = agg_ref.shape[0]
    nin = nin_ref[...]
    w = w_ref[...]
    b = b_ref[...]
    for t in range(T):
        a = (agg_ref[t] + h1_ref[t]) * nin
        out_ref[:, t, :] = jnp.dot(
            a, w, preferred_element_type=jnp.float32) + b


def _c2(agg1, h1, nin1, w, b):
    t, n, d = agg1.shape
    h = w.shape[1]
    nb = n // _BLK
    return pl.pallas_call(
        _c2_body,
        grid=(nb,),
        in_specs=[
            pl.BlockSpec((t, _BLK, d), lambda i: (0, i, 0)),
            pl.BlockSpec((t, _BLK, d), lambda i: (0, i, 0)),
            pl.BlockSpec((_BLK, 1), lambda i: (i, 0)),
            pl.BlockSpec((d, h), lambda i: (0, 0)),
            pl.BlockSpec((1, h), lambda i: (0, 0)),
        ],
        out_specs=pl.BlockSpec((_BLK, t, h), lambda i: (i, 0, 0)),
        out_shape=jax.ShapeDtypeStruct((n, t, h), jnp.float32),
    )(agg1, h1, nin1, w, b)


_CHUNK = 500
_DN = (((1,), (1,)), ((), ()))
_PREC = lax.Precision.HIGHEST


def _gru_body(x_ref, wi0, whc, wi1, bi0, bhc, bi1,
              out_ref, gi_ref, hs_ref):
    chunk, T, D = x_ref.shape
    H = D
    c = pl.program_id(0)
    nc = pl.num_programs(0)

    @pl.when(c == 0)
    def _():
        hs_ref[...] = jnp.zeros_like(hs_ref)

    x = x_ref[...].reshape(chunk * T, D)
    gi_ref[...] = lax.dot_general(
        x, wi0[...], _DN, preferred_element_type=jnp.float32,
        precision=_PREC) + bi0[...]

    def step2(i, _):
        base = pl.multiple_of(8 * i, 8)
        g8 = gi_ref[pl.ds(base, 8), :]
        for k in range(2):
            g0 = g8[4 * k:4 * k + 4, :]
            hs = hs_ref[...]
            h0 = hs[:, 0:H]
            h1 = hs[:, H:2 * H]
            gh = jnp.dot(hs, whc[...], preferred_element_type=jnp.float32,
                         precision=_PREC) + bhc[...]
            r0 = jax.nn.sigmoid(g0[:, 0:H] + gh[:, 0:H])
            z0 = jax.nn.sigmoid(g0[:, H:2 * H] + gh[:, H:2 * H])
            n0 = jnp.tanh(g0[:, 2 * H:3 * H] + r0 * gh[:, 2 * H:3 * H])
            h0n = n0 + z0 * (h0 - n0)
            gi1 = lax.dot_general(
                h0n, wi1[...], _DN, preferred_element_type=jnp.float32,
                precision=_PREC) + bi1[...]
            r1 = jax.nn.sigmoid(gi1[:, 0:H] + gh[:, 3 * H:4 * H])
            z1 = jax.nn.sigmoid(gi1[:, H:2 * H] + gh[:, 4 * H:5 * H])
            n1 = jnp.tanh(gi1[:, 2 * H:3 * H] + r1 * gh[:, 5 * H:6 * H])
            h1n = n1 + z1 * (h1 - n1)
            hs_ref[:, 0:H] = h0n
            hs_ref[:, H:2 * H] = h1n
        return 0

    lax.fori_loop(0, chunk // 2, step2, 0)

    @pl.when(c == nc - 1)
    def _():
        out_ref[...] = jnp.maximum(hs_ref[:, H:2 * H], 0.0)


def _gru(x, wi0, whc, wi1, bi0, bhc, bi1):
    n, t, d = x.shape
    h3 = wi0.shape[0]
    nc = n // _CHUNK
    return pl.pallas_call(
        _gru_body,
        grid=(nc,),
        in_specs=[
            pl.BlockSpec((_CHUNK, t, d), lambda c: (c, 0, 0)),
            pl.BlockSpec(wi0.shape, lambda c: (0, 0)),
            pl.BlockSpec(whc.shape, lambda c: (0, 0)),
            pl.BlockSpec(wi1.shape, lambda c: (0, 0)),
            pl.BlockSpec(bi0.shape, lambda c: (0, 0)),
            pl.BlockSpec(bhc.shape, lambda c: (0, 0)),
            pl.BlockSpec(bi1.shape, lambda c: (0, 0)),
        ],
        out_specs=pl.BlockSpec((t, d), lambda c: (0, 0)),
        out_shape=jax.ShapeDtypeStruct((t, d), jnp.float32),
        scratch_shapes=[
            pltpu.VMEM((_CHUNK * t, h3), jnp.float32),
            pltpu.VMEM((t, 2 * d), jnp.float32),
        ],
    )(x, wi0, whc, wi1, bi0, bhc, bi1)


def kernel(features, edge_index, W_conv, b_conv, w_ih0, w_hh0, b_ih0, b_hh0,
           w_ih1, w_hh1, b_ih1, b_hh1):
    t, n, d = features.shape
    h = W_conv.shape[1]
    src = edge_index[0]
    dst = edge_index[1]
    degs = _degrees(src, dst, n)
    h0, nin0, nout1, nin1 = _c0(features, degs)
    agg0 = _segsum(h0.reshape(t * n, d), src, dst, n)
    h1 = _c1(agg0.reshape(t, n, d), nin0, nout1, W_conv, b_conv.reshape(1, h))
    agg1 = _segsum(h1.reshape(t * n, h), src, dst, n)
    x2 = _c2(agg1.reshape(t, n, h), h1, nin1, W_conv, b_conv.reshape(1, h))
    whc = jnp.zeros((2 * h, 6 * h), jnp.float32)
    whc = whc.at[0:h, 0:3 * h].set(w_hh0.T)
    whc = whc.at[h:2 * h, 3 * h:6 * h].set(w_hh1.T)
    bhc = jnp.concatenate([b_hh0, b_hh1]).reshape(1, -1)
    out = _gru(x2, w_ih0, whc, w_ih1,
               b_ih0.reshape(1, -1), bhc, b_ih1.reshape(1, -1))
    return out[:, None, :]


# speedup vs baseline: 2.3126x; 1.2695x over previous
import functools

import jax
import jax.numpy as jnp
from jax import lax
from jax.experimental import pallas as pl
from jax.experimental.pallas import tpu as pltpu
from jax.experimental.pallas import tpu_sc as plsc

_NC = 2
_NS = 16


_DEG_K = 80
_DEG_ZR = 128
_DEG_STRIDE = 624


def _degree_body(sd_hbm, out_hbm, didx, ones_rows, zbuf, acc, sem):
    c = lax.axis_index("c")
    s = lax.axis_index("s")
    E = sd_hbm.shape[0] // 2
    K = _DEG_K
    epw = E // _NS
    nch = epw // K
    one = jnp.ones((16,), jnp.float32)
    zv = jnp.zeros((16,), jnp.float32)

    def fill(i, _):
        zbuf[i // 8, pl.ds((i % 8) * 16, 16)] = zv
        ones_rows[i // 8, pl.ds((i % 8) * 16, 16)] = one
        return 0

    lax.fori_loop(0, _DEG_ZR * 8, fill, 0)
    rbase = s * _DEG_STRIDE
    for z in range(5):
        pltpu.sync_copy(zbuf, acc.at[pl.ds(rbase + z * _DEG_ZR, _DEG_ZR)])
    plsc.subcore_barrier()

    def chunk(ch, _):
        ebase = c * E + s * epw + ch * K
        pltpu.sync_copy(sd_hbm.at[pl.ds(ebase, K)], didx)
        pltpu.sync_copy(ones_rows.at[pl.ds(0, K)], acc.at[didx], add=True)
        return 0

    lax.fori_loop(0, nch, chunk, 0)
    plsc.subcore_barrier()
    for z in range(5):
        rb = rbase + z * _DEG_ZR
        pltpu.sync_copy(acc.at[pl.ds(rb, _DEG_ZR)],
                        out_hbm.at[c, pl.ds(rb, _DEG_ZR)])


def _degrees(src, dst, n):
    d = 128
    sd = jnp.concatenate([src, dst])
    return pl.kernel(
        _degree_body,
        out_type=jax.ShapeDtypeStruct((2, n, d), jnp.float32),
        mesh=plsc.VectorSubcoreMesh(core_axis_name="c", subcore_axis_name="s",
                                    num_cores=_NC, num_subcores=_NS),
        scratch_types=[
            pltpu.VMEM((_DEG_K,), jnp.int32),
            pltpu.VMEM((_DEG_ZR, d), jnp.float32),
            pltpu.VMEM((_DEG_ZR, d), jnp.float32),
            pltpu.VMEM_SHARED((n, d), jnp.float32),
            pltpu.SemaphoreType.DMA,
        ],
    )(sd)


_SEG_K = 80
_SEG_ZR = 128
_SEG_STRIDE = 624


def _segsum_body(h_hbm, src_hbm, dst_hbm, out_hbm, sidx2, didx2, rows2,
                 zbuf, acc, sem0, sem1):
    c = lax.axis_index("c")
    s = lax.axis_index("s")
    E = src_hbm.shape[0]
    N = acc.shape[0]
    K = _SEG_K
    epw = E // _NS
    nch = epw // K
    sems = (sem0, sem1)
    zv = jnp.zeros((16,), jnp.float32)

    def zb(i, _):
        zbuf[i // 8, pl.ds((i % 8) * 16, 16)] = zv
        return 0

    lax.fori_loop(0, _SEG_ZR * 8, zb, 0)
    rbase = s * _SEG_STRIDE

    for t_local in range(2):
        toff = (c * 2 + t_local) * N
        tvec = jnp.full((16,), toff, jnp.int32)
        for z in range(5):
            pltpu.sync_copy(zbuf, acc.at[pl.ds(rbase + z * _SEG_ZR, _SEG_ZR)])
        plsc.subcore_barrier()

        def load_and_gather(ch, b):
            ebase = s * epw + ch * K
            pltpu.sync_copy(src_hbm.at[pl.ds(ebase, K)], sidx2.at[b])
            for j in range(K // 16):
                sidx2[b, pl.ds(j * 16, 16)] = \
                    sidx2[b, pl.ds(j * 16, 16)] + tvec
            pltpu.sync_copy(dst_hbm.at[pl.ds(ebase, K)], didx2.at[b])
            pltpu.async_copy(h_hbm.at[sidx2.at[b]], rows2.at[b], sems[b])

        def drain(b):
            pltpu.make_async_copy(
                h_hbm.at[sidx2.at[b]], rows2.at[b], sems[b]).wait()

        for b in range(2):
            load_and_gather(b, b)

        def pair(i, _):
            for b in range(2):
                ch = 2 * i + b
                drain(b)
                pltpu.sync_copy(rows2.at[b], acc.at[didx2.at[b]], add=True)
                load_and_gather(ch + 2, b)
            return 0

        lax.fori_loop(0, nch // 2 - 1, pair, 0)
        for b in range(2):
            drain(b)
            pltpu.sync_copy(rows2.at[b], acc.at[didx2.at[b]], add=True)
        plsc.subcore_barrier()
        for z in range(5):
            rb = rbase + z * _SEG_ZR
            pltpu.sync_copy(acc.at[pl.ds(rb, _SEG_ZR)],
                            out_hbm.at[pl.ds(toff + rb, _SEG_ZR)])
        plsc.subcore_barrier()


def _segsum(h2, src, dst, n):
    tn, d = h2.shape
    return pl.kernel(
        _segsum_body,
        out_type=jax.ShapeDtypeStruct((tn, d), jnp.float32),
        mesh=plsc.VectorSubcoreMesh(core_axis_name="c", subcore_axis_name="s",
                                    num_cores=_NC, num_subcores=_NS),
        scratch_types=[
            pltpu.VMEM((2, _SEG_K), jnp.int32),
            pltpu.VMEM((2, _SEG_K), jnp.int32),
            pltpu.VMEM((2, _SEG_K, d), jnp.float32),
            pltpu.VMEM((_SEG_ZR, d), jnp.float32),
            pltpu.VMEM_SHARED((n, d), jnp.float32),
            pltpu.SemaphoreType.DMA,
            pltpu.SemaphoreType.DMA,
        ],
    )(h2, src, dst)


_BLK = 2000


def _c0_body(x_ref, degs_ref, h_ref, nin0_ref, nout1_ref, nin1_ref):
    T = x_ref.shape[0]
    deg_out = degs_ref[0, :, 0:1]
    deg_in = degs_ref[1, :, 0:1]
    nout0 = lax.rsqrt(jnp.maximum(deg_out, 1.0))
    nin0_ref[...] = lax.rsqrt(jnp.maximum(deg_in, 1.0))
    nout1_ref[...] = lax.rsqrt(jnp.maximum(deg_out + 1.0, 1.0))
    nin1_ref[...] = lax.rsqrt(jnp.maximum(deg_in + 1.0, 1.0))
    for t in range(T):
        h_ref[t] = x_ref[t] * nout0


def _c0(features, degs):
    t, n, d = features.shape
    nb = n // _BLK
    return pl.pallas_call(
        _c0_body,
        grid=(nb,),
        in_specs=[
            pl.BlockSpec((t, _BLK, d), lambda i: (0, i, 0)),
            pl.BlockSpec((2, _BLK, 128), lambda i: (0, i, 0)),
        ],
        out_specs=[
            pl.BlockSpec((t, _BLK, d), lambda i: (0, i, 0)),
            pl.BlockSpec((_BLK, 1), lambda i: (i, 0)),
            pl.BlockSpec((_BLK, 1), lambda i: (i, 0)),
            pl.BlockSpec((_BLK, 1), lambda i: (i, 0)),
        ],
        out_shape=[
            jax.ShapeDtypeStruct((t, n, d), jnp.float32),
            jax.ShapeDtypeStruct((n, 1), jnp.float32),
            jax.ShapeDtypeStruct((n, 1), jnp.float32),
            jax.ShapeDtypeStruct((n, 1), jnp.float32),
        ],
    )(features, degs)


def _c1_body(agg_ref, nin_ref, nout_ref, w_ref, b_ref, h_ref):
    a = agg_ref[0] * nin_ref[...]
    x1 = jnp.dot(a, w_ref[...], preferred_element_type=jnp.float32) + b_ref[...]
    h_ref[0] = x1 * nout_ref[...]


def _c1(agg0, nin0, nout1, w, b):
    t, n, d = agg0.shape
    h = w.shape[1]
    nb = n // _BLK
    return pl.pallas_call(
        _c1_body,
        grid=(t, nb),
        in_specs=[
            pl.BlockSpec((1, _BLK, d), lambda ti, i: (ti, i, 0)),
            pl.BlockSpec((_BLK, 1), lambda ti, i: (i, 0)),
            pl.BlockSpec((_BLK, 1), lambda ti, i: (i, 0)),
            pl.BlockSpec((d, h), lambda ti, i: (0, 0)),
            pl.BlockSpec((1, h), lambda ti, i: (0, 0)),
        ],
        out_specs=pl.BlockSpec((1, _BLK, h), lambda ti, i: (ti, i, 0)),
        out_shape=jax.ShapeDtypeStruct((t, n, h), jnp.float32),
    )(agg0, nin0, nout1, w, b)


def _c2_body(agg_ref, h1_ref, nin_ref, w_ref, b_ref, out_ref):
    T = agg_ref.shape[0]
    nin = nin_ref[...]
    w = w_ref[...]
    b = b_ref[...]
    for t in range(T):
        a = (agg_ref[t] + h1_ref[t]) * nin
        out_ref[:, t, :] = jnp.dot(
            a, w, preferred_element_type=jnp.float32) + b


def _c2(agg1, h1, nin1, w, b):
    t, n, d = agg1.shape
    h = w.shape[1]
    nb = n // _BLK
    return pl.pallas_call(
        _c2_body,
        grid=(nb,),
        in_specs=[
            pl.BlockSpec((t, _BLK, d), lambda i: (0, i, 0)),
            pl.BlockSpec((t, _BLK, d), lambda i: (0, i, 0)),
            pl.BlockSpec((_BLK, 1), lambda i: (i, 0)),
            pl.BlockSpec((d, h), lambda i: (0, 0)),
            pl.BlockSpec((1, h), lambda i: (0, 0)),
        ],
        out_specs=pl.BlockSpec((_BLK, t, h), lambda i: (i, 0, 0)),
        out_shape=jax.ShapeDtypeStruct((n, t, h), jnp.float32),
    )(agg1, h1, nin1, w, b)


_CHUNK = 500
_DN = (((1,), (1,)), ((), ()))
_PREC = lax.Precision.HIGHEST


def _gru_body(x_ref, wi0, whc, wi1, bi0, bhc, bi1,
              out_ref, gi_ref, hs_ref):
    chunk, T, D = x_ref.shape
    H = D
    c = pl.program_id(0)
    nc = pl.num_programs(0)

    @pl.when(c == 0)
    def _():
        hs_ref[...] = jnp.zeros_like(hs_ref)

    x = x_ref[...].reshape(chunk * T, D)
    gi_ref[...] = lax.dot_general(
        x, wi0[...], _DN, preferred_element_type=jnp.float32,
        precision=lax.Precision.DEFAULT) + bi0[...]

    def step2(i, _):
        base = pl.multiple_of(8 * i, 8)
        g8 = gi_ref[pl.ds(base, 8), :]
        for k in range(2):
            g0 = g8[4 * k:4 * k + 4, :]
            hs = hs_ref[...]
            h0 = hs[:, 0:H]
            h1 = hs[:, H:2 * H]
            gh = jnp.dot(hs, whc[...], preferred_element_type=jnp.float32,
                         precision=_PREC) + bhc[...]
            r0 = jax.nn.sigmoid(g0[:, 0:H] + gh[:, 0:H])
            z0 = jax.nn.sigmoid(g0[:, H:2 * H] + gh[:, H:2 * H])
            n0 = jnp.tanh(g0[:, 2 * H:3 * H] + r0 * gh[:, 2 * H:3 * H])
            h0n = n0 + z0 * (h0 - n0)
            gi1 = lax.dot_general(
                h0n, wi1[...], _DN, preferred_element_type=jnp.float32,
                precision=lax.Precision.DEFAULT) + bi1[...]
            r1 = jax.nn.sigmoid(gi1[:, 0:H] + gh[:, 3 * H:4 * H])
            z1 = jax.nn.sigmoid(gi1[:, H:2 * H] + gh[:, 4 * H:5 * H])
            n1 = jnp.tanh(gi1[:, 2 * H:3 * H] + r1 * gh[:, 5 * H:6 * H])
            h1n = n1 + z1 * (h1 - n1)
            hs_ref[:, 0:H] = h0n
            hs_ref[:, H:2 * H] = h1n
        return 0

    lax.fori_loop(0, chunk // 2, step2, 0)

    @pl.when(c == nc - 1)
    def _():
        out_ref[...] = jnp.maximum(hs_ref[:, H:2 * H], 0.0)


def _gru(x, wi0, whc, wi1, bi0, bhc, bi1):
    n, t, d = x.shape
    h3 = wi0.shape[0]
    nc = n // _CHUNK
    return pl.pallas_call(
        _gru_body,
        grid=(nc,),
        in_specs=[
            pl.BlockSpec((_CHUNK, t, d), lambda c: (c, 0, 0)),
            pl.BlockSpec(wi0.shape, lambda c: (0, 0)),
            pl.BlockSpec(whc.shape, lambda c: (0, 0)),
            pl.BlockSpec(wi1.shape, lambda c: (0, 0)),
            pl.BlockSpec(bi0.shape, lambda c: (0, 0)),
            pl.BlockSpec(bhc.shape, lambda c: (0, 0)),
            pl.BlockSpec(bi1.shape, lambda c: (0, 0)),
        ],
        out_specs=pl.BlockSpec((t, d), lambda c: (0, 0)),
        out_shape=jax.ShapeDtypeStruct((t, d), jnp.float32),
        scratch_shapes=[
            pltpu.VMEM((_CHUNK * t, h3), jnp.float32),
            pltpu.VMEM((t, 2 * d), jnp.float32),
        ],
    )(x, wi0, whc, wi1, bi0, bhc, bi1)


def kernel(features, edge_index, W_conv, b_conv, w_ih0, w_hh0, b_ih0, b_hh0,
           w_ih1, w_hh1, b_ih1, b_hh1):
    t, n, d = features.shape
    h = W_conv.shape[1]
    src = edge_index[0]
    dst = edge_index[1]
    degs = _degrees(src, dst, n)
    h0, nin0, nout1, nin1 = _c0(features, degs)
    agg0 = _segsum(h0.reshape(t * n, d), src, dst, n)
    h1 = _c1(agg0.reshape(t, n, d), nin0, nout1, W_conv, b_conv.reshape(1, h))
    agg1 = _segsum(h1.reshape(t * n, h), src, dst, n)
    x2 = _c2(agg1.reshape(t, n, h), h1, nin1, W_conv, b_conv.reshape(1, h))
    whc = jnp.zeros((2 * h, 6 * h), jnp.float32)
    whc = whc.at[0:h, 0:3 * h].set(w_hh0.T)
    whc = whc.at[h:2 * h, 3 * h:6 * h].set(w_hh1.T)
    bhc = jnp.concatenate([b_hh0, b_hh1]).reshape(1, -1)
    out = _gru(x2, w_ih0, whc, w_ih1,
               b_ih0.reshape(1, -1), bhc, b_ih1.reshape(1, -1))
    return out[:, None, :]


# speedup vs baseline: 3.5663x; 1.5421x over previous
import functools

import jax
import jax.numpy as jnp
from jax import lax
from jax.experimental import pallas as pl
from jax.experimental.pallas import tpu as pltpu
from jax.experimental.pallas import tpu_sc as plsc

_NC = 2
_NS = 16


_DEG_K = 80
_DEG_ZR = 128
_DEG_STRIDE = 624


def _degree_body(sd_hbm, out_hbm, didx, ones_rows, zbuf, acc, sem):
    c = lax.axis_index("c")
    s = lax.axis_index("s")
    E = sd_hbm.shape[0] // 2
    K = _DEG_K
    epw = E // _NS
    nch = epw // K
    one = jnp.ones((16,), jnp.float32)
    zv = jnp.zeros((16,), jnp.float32)

    def fill(i, _):
        zbuf[i // 8, pl.ds((i % 8) * 16, 16)] = zv
        ones_rows[i // 8, pl.ds((i % 8) * 16, 16)] = one
        return 0

    lax.fori_loop(0, _DEG_ZR * 8, fill, 0)
    rbase = s * _DEG_STRIDE
    for z in range(5):
        pltpu.sync_copy(zbuf, acc.at[pl.ds(rbase + z * _DEG_ZR, _DEG_ZR)])
    plsc.subcore_barrier()

    def chunk(ch, _):
        ebase = c * E + s * epw + ch * K
        pltpu.sync_copy(sd_hbm.at[pl.ds(ebase, K)], didx)
        pltpu.sync_copy(ones_rows.at[pl.ds(0, K)], acc.at[didx], add=True)
        return 0

    lax.fori_loop(0, nch, chunk, 0)
    plsc.subcore_barrier()
    for z in range(5):
        rb = rbase + z * _DEG_ZR
        pltpu.sync_copy(acc.at[pl.ds(rb, _DEG_ZR)],
                        out_hbm.at[c, pl.ds(rb, _DEG_ZR)])


def _degrees(src, dst, n):
    d = 128
    sd = jnp.concatenate([src, dst])
    return pl.kernel(
        _degree_body,
        out_type=jax.ShapeDtypeStruct((2, n, d), jnp.float32),
        mesh=plsc.VectorSubcoreMesh(core_axis_name="c", subcore_axis_name="s",
                                    num_cores=_NC, num_subcores=_NS),
        scratch_types=[
            pltpu.VMEM((_DEG_K,), jnp.int32),
            pltpu.VMEM((_DEG_ZR, d), jnp.float32),
            pltpu.VMEM((_DEG_ZR, d), jnp.float32),
            pltpu.VMEM_SHARED((n, d), jnp.float32),
            pltpu.SemaphoreType.DMA,
        ],
    )(sd)


_SEG_K = 80
_SEG_ZR = 128
_SEG_STRIDE = 624


def _segsum_body(h_hbm, src_hbm, dst_hbm, out_hbm, sidx2, didx2, rows2,
                 zbuf, acc, sem0, sem1):
    c = lax.axis_index("c")
    s = lax.axis_index("s")
    E = src_hbm.shape[0]
    N = acc.shape[0]
    K = _SEG_K
    epw = E // _NS
    nch = epw // K
    sems = (sem0, sem1)
    zv = jnp.zeros((16,), jnp.float32)

    def zb(i, _):
        zbuf[i // 8, pl.ds((i % 8) * 16, 16)] = zv
        return 0

    lax.fori_loop(0, _SEG_ZR * 8, zb, 0)
    rbase = s * _SEG_STRIDE

    for t_local in range(2):
        toff = (c * 2 + t_local) * N
        tvec = jnp.full((16,), toff, jnp.int32)
        for z in range(5):
            pltpu.sync_copy(zbuf, acc.at[pl.ds(rbase + z * _SEG_ZR, _SEG_ZR)])
        plsc.subcore_barrier()

        def load_and_gather(ch, b):
            ebase = s * epw + ch * K
            pltpu.sync_copy(src_hbm.at[pl.ds(ebase, K)], sidx2.at[b])
            for j in range(K // 16):
                sidx2[b, pl.ds(j * 16, 16)] = \
                    sidx2[b, pl.ds(j * 16, 16)] + tvec
            pltpu.sync_copy(dst_hbm.at[pl.ds(ebase, K)], didx2.at[b])
            pltpu.async_copy(h_hbm.at[sidx2.at[b]], rows2.at[b], sems[b])

        def drain(b):
            pltpu.make_async_copy(
                h_hbm.at[sidx2.at[b]], rows2.at[b], sems[b]).wait()

        for b in range(2):
            load_and_gather(b, b)

        def pair(i, _):
            for b in range(2):
                ch = 2 * i + b
                drain(b)
                pltpu.sync_copy(rows2.at[b], acc.at[didx2.at[b]], add=True)
                load_and_gather(ch + 2, b)
            return 0

        lax.fori_loop(0, nch // 2 - 1, pair, 0)
        for b in range(2):
            drain(b)
            pltpu.sync_copy(rows2.at[b], acc.at[didx2.at[b]], add=True)
        plsc.subcore_barrier()
        for z in range(5):
            rb = rbase + z * _SEG_ZR
            pltpu.sync_copy(acc.at[pl.ds(rb, _SEG_ZR)],
                            out_hbm.at[pl.ds(toff + rb, _SEG_ZR)])
        plsc.subcore_barrier()


def _segsum(h2, src, dst, n):
    tn, d = h2.shape
    return pl.kernel(
        _segsum_body,
        out_type=jax.ShapeDtypeStruct((tn, d), jnp.float32),
        mesh=plsc.VectorSubcoreMesh(core_axis_name="c", subcore_axis_name="s",
                                    num_cores=_NC, num_subcores=_NS),
        scratch_types=[
            pltpu.VMEM((2, _SEG_K), jnp.int32),
            pltpu.VMEM((2, _SEG_K), jnp.int32),
            pltpu.VMEM((2, _SEG_K, d), jnp.float32),
            pltpu.VMEM((_SEG_ZR, d), jnp.float32),
            pltpu.VMEM_SHARED((n, d), jnp.float32),
            pltpu.SemaphoreType.DMA,
            pltpu.SemaphoreType.DMA,
        ],
    )(h2, src, dst)


_BLK = 2000


def _c0_body(x_ref, degs_ref, h_ref, nin0_ref, nout1_ref, nin1_ref):
    T = x_ref.shape[0]
    deg_out = degs_ref[0, :, 0:1]
    deg_in = degs_ref[1, :, 0:1]
    nout0 = lax.rsqrt(jnp.maximum(deg_out, 1.0))
    nin0_ref[...] = lax.rsqrt(jnp.maximum(deg_in, 1.0))
    nout1_ref[...] = lax.rsqrt(jnp.maximum(deg_out + 1.0, 1.0))
    nin1_ref[...] = lax.rsqrt(jnp.maximum(deg_in + 1.0, 1.0))
    for t in range(T):
        h_ref[t] = x_ref[t] * nout0


def _c0(features, degs):
    t, n, d = features.shape
    nb = n // _BLK
    return pl.pallas_call(
        _c0_body,
        grid=(nb,),
        in_specs=[
            pl.BlockSpec((t, _BLK, d), lambda i: (0, i, 0)),
            pl.BlockSpec((2, _BLK, 128), lambda i: (0, i, 0)),
        ],
        out_specs=[
            pl.BlockSpec((t, _BLK, d), lambda i: (0, i, 0)),
            pl.BlockSpec((_BLK, 1), lambda i: (i, 0)),
            pl.BlockSpec((_BLK, 1), lambda i: (i, 0)),
            pl.BlockSpec((_BLK, 1), lambda i: (i, 0)),
        ],
        out_shape=[
            jax.ShapeDtypeStruct((t, n, d), jnp.float32),
            jax.ShapeDtypeStruct((n, 1), jnp.float32),
            jax.ShapeDtypeStruct((n, 1), jnp.float32),
            jax.ShapeDtypeStruct((n, 1), jnp.float32),
        ],
    )(features, degs)


def _c1_body(agg_ref, nin_ref, nout_ref, w_ref, b_ref, h_ref):
    a = agg_ref[0] * nin_ref[...]
    x1 = jnp.dot(a, w_ref[...], preferred_element_type=jnp.float32) + b_ref[...]
    h_ref[0] = x1 * nout_ref[...]


def _c1(agg0, nin0, nout1, w, b):
    t, n, d = agg0.shape
    h = w.shape[1]
    nb = n // _BLK
    return pl.pallas_call(
        _c1_body,
        grid=(t, nb),
        in_specs=[
            pl.BlockSpec((1, _BLK, d), lambda ti, i: (ti, i, 0)),
            pl.BlockSpec((_BLK, 1), lambda ti, i: (i, 0)),
            pl.BlockSpec((_BLK, 1), lambda ti, i: (i, 0)),
            pl.BlockSpec((d, h), lambda ti, i: (0, 0)),
            pl.BlockSpec((1, h), lambda ti, i: (0, 0)),
        ],
        out_specs=pl.BlockSpec((1, _BLK, h), lambda ti, i: (ti, i, 0)),
        out_shape=jax.ShapeDtypeStruct((t, n, h), jnp.float32),
    )(agg0, nin0, nout1, w, b)


def _c2_body(agg_ref, h1_ref, nin_ref, w_ref, b_ref, out_ref):
    T = agg_ref.shape[0]
    nin = nin_ref[...]
    w = w_ref[...]
    b = b_ref[...]
    for t in range(T):
        a = (agg_ref[t] + h1_ref[t]) * nin
        out_ref[:, t, :] = jnp.dot(
            a, w, preferred_element_type=jnp.float32) + b


def _c2(agg1, h1, nin1, w, b):
    t, n, d = agg1.shape
    h = w.shape[1]
    nb = n // _BLK
    return pl.pallas_call(
        _c2_body,
        grid=(nb,),
        in_specs=[
            pl.BlockSpec((t, _BLK, d), lambda i: (0, i, 0)),
            pl.BlockSpec((t, _BLK, d), lambda i: (0, i, 0)),
            pl.BlockSpec((_BLK, 1), lambda i: (i, 0)),
            pl.BlockSpec((d, h), lambda i: (0, 0)),
            pl.BlockSpec((1, h), lambda i: (0, 0)),
        ],
        out_specs=pl.BlockSpec((_BLK, t, h), lambda i: (i, 0, 0)),
        out_shape=jax.ShapeDtypeStruct((n, t, h), jnp.float32),
    )(agg1, h1, nin1, w, b)


_CHUNK = 500
_DN = (((1,), (1,)), ((), ()))
_PREC = lax.Precision.DEFAULT


def _gru_body(x_ref, wi0, whc, wi1, bi0, bhc, bi1,
              out_ref, gi_ref, hs_ref):
    chunk, T, D = x_ref.shape
    H = D
    c = pl.program_id(0)
    nc = pl.num_programs(0)

    @pl.when(c == 0)
    def _():
        hs_ref[...] = jnp.zeros_like(hs_ref)

    x = x_ref[...].reshape(chunk * T, D)
    gi_ref[...] = lax.dot_general(
        x, wi0[...], _DN, preferred_element_type=jnp.float32,
        precision=lax.Precision.DEFAULT) + bi0[...]

    def step2(i, _):
        base = pl.multiple_of(8 * i, 8)
        g8 = gi_ref[pl.ds(base, 8), :]
        for k in range(2):
            g0 = g8[4 * k:4 * k + 4, :]
            hs = hs_ref[...]
            h0 = hs[:, 0:H]
            h1 = hs[:, H:2 * H]
            gh = jnp.dot(hs, whc[...], preferred_element_type=jnp.float32,
                         precision=_PREC) + bhc[...]
            r0 = jax.nn.sigmoid(g0[:, 0:H] + gh[:, 0:H])
            z0 = jax.nn.sigmoid(g0[:, H:2 * H] + gh[:, H:2 * H])
            n0 = jnp.tanh(g0[:, 2 * H:3 * H] + r0 * gh[:, 2 * H:3 * H])
            h0n = n0 + z0 * (h0 - n0)
            gi1 = lax.dot_general(
                h0n, wi1[...], _DN, preferred_element_type=jnp.float32,
                precision=lax.Precision.DEFAULT) + bi1[...]
            r1 = jax.nn.sigmoid(gi1[:, 0:H] + gh[:, 3 * H:4 * H])
            z1 = jax.nn.sigmoid(gi1[:, H:2 * H] + gh[:, 4 * H:5 * H])
            n1 = jnp.tanh(gi1[:, 2 * H:3 * H] + r1 * gh[:, 5 * H:6 * H])
            h1n = n1 + z1 * (h1 - n1)
            hs_ref[:, 0:H] = h0n
            hs_ref[:, H:2 * H] = h1n
        return 0

    lax.fori_loop(0, chunk // 2, step2, 0)

    @pl.when(c == nc - 1)
    def _():
        out_ref[...] = jnp.maximum(hs_ref[:, H:2 * H], 0.0)


def _gru(x, wi0, whc, wi1, bi0, bhc, bi1):
    n, t, d = x.shape
    h3 = wi0.shape[0]
    nc = n // _CHUNK
    return pl.pallas_call(
        _gru_body,
        grid=(nc,),
        in_specs=[
            pl.BlockSpec((_CHUNK, t, d), lambda c: (c, 0, 0)),
            pl.BlockSpec(wi0.shape, lambda c: (0, 0)),
            pl.BlockSpec(whc.shape, lambda c: (0, 0)),
            pl.BlockSpec(wi1.shape, lambda c: (0, 0)),
            pl.BlockSpec(bi0.shape, lambda c: (0, 0)),
            pl.BlockSpec(bhc.shape, lambda c: (0, 0)),
            pl.BlockSpec(bi1.shape, lambda c: (0, 0)),
        ],
        out_specs=pl.BlockSpec((t, d), lambda c: (0, 0)),
        out_shape=jax.ShapeDtypeStruct((t, d), jnp.float32),
        scratch_shapes=[
            pltpu.VMEM((_CHUNK * t, h3), jnp.float32),
            pltpu.VMEM((t, 2 * d), jnp.float32),
        ],
    )(x, wi0, whc, wi1, bi0, bhc, bi1)


def kernel(features, edge_index, W_conv, b_conv, w_ih0, w_hh0, b_ih0, b_hh0,
           w_ih1, w_hh1, b_ih1, b_hh1):
    t, n, d = features.shape
    h = W_conv.shape[1]
    src = edge_index[0]
    dst = edge_index[1]
    degs = _degrees(src, dst, n)
    h0, nin0, nout1, nin1 = _c0(features, degs)
    agg0 = _segsum(h0.reshape(t * n, d), src, dst, n)
    h1 = _c1(agg0.reshape(t, n, d), nin0, nout1, W_conv, b_conv.reshape(1, h))
    agg1 = _segsum(h1.reshape(t * n, h), src, dst, n)
    x2 = _c2(agg1.reshape(t, n, h), h1, nin1, W_conv, b_conv.reshape(1, h))
    whc = jnp.zeros((2 * h, 6 * h), jnp.float32)
    whc = whc.at[0:h, 0:3 * h].set(w_hh0.T)
    whc = whc.at[h:2 * h, 3 * h:6 * h].set(w_hh1.T)
    bhc = jnp.concatenate([b_hh0, b_hh1]).reshape(1, -1)
    out = _gru(x2, w_ih0, whc, w_ih1,
               b_ih0.reshape(1, -1), bhc, b_ih1.reshape(1, -1))
    return out[:, None, :]


# speedup vs baseline: 4.0137x; 1.1255x over previous
import functools

import jax
import jax.numpy as jnp
from jax import lax
from jax.experimental import pallas as pl
from jax.experimental.pallas import tpu as pltpu
from jax.experimental.pallas import tpu_sc as plsc

_NC = 2
_NS = 16


_DEG_K = 80
_DEG_ZR = 128
_DEG_STRIDE = 624


def _degree_body(sd_hbm, out_hbm, didx, ones_rows, zbuf, acc, sem):
    c = lax.axis_index("c")
    s = lax.axis_index("s")
    E = sd_hbm.shape[0] // 2
    K = _DEG_K
    epw = E // _NS
    nch = epw // K
    one = jnp.ones((16,), jnp.float32)
    zv = jnp.zeros((16,), jnp.float32)

    def fill(i, _):
        zbuf[i // 8, pl.ds((i % 8) * 16, 16)] = zv
        ones_rows[i // 8, pl.ds((i % 8) * 16, 16)] = one
        return 0

    lax.fori_loop(0, _DEG_ZR * 8, fill, 0)
    rbase = s * _DEG_STRIDE
    for z in range(5):
        pltpu.sync_copy(zbuf, acc.at[pl.ds(rbase + z * _DEG_ZR, _DEG_ZR)])
    plsc.subcore_barrier()

    def chunk(ch, _):
        ebase = c * E + s * epw + ch * K
        pltpu.sync_copy(sd_hbm.at[pl.ds(ebase, K)], didx)
        pltpu.sync_copy(ones_rows.at[pl.ds(0, K)], acc.at[didx], add=True)
        return 0

    lax.fori_loop(0, nch, chunk, 0)
    plsc.subcore_barrier()
    for z in range(5):
        rb = rbase + z * _DEG_ZR
        pltpu.sync_copy(acc.at[pl.ds(rb, _DEG_ZR)],
                        out_hbm.at[c, pl.ds(rb, _DEG_ZR)])


def _degrees(src, dst, n):
    d = 128
    sd = jnp.concatenate([src, dst])
    return pl.kernel(
        _degree_body,
        out_type=jax.ShapeDtypeStruct((2, n, d), jnp.float32),
        mesh=plsc.VectorSubcoreMesh(core_axis_name="c", subcore_axis_name="s",
                                    num_cores=_NC, num_subcores=_NS),
        scratch_types=[
            pltpu.VMEM((_DEG_K,), jnp.int32),
            pltpu.VMEM((_DEG_ZR, d), jnp.float32),
            pltpu.VMEM((_DEG_ZR, d), jnp.float32),
            pltpu.VMEM_SHARED((n, d), jnp.float32),
            pltpu.SemaphoreType.DMA,
        ],
    )(sd)


_SEG_K = 80
_SEG_ZR = 128
_SEG_STRIDE = 624


def _segsum_body(h_hbm, src_hbm, dst_hbm, out_hbm, sidx4, didx4, rows2,
                 zbuf, acc, gs0, gs1, is0, is1, is2, is3):
    c = lax.axis_index("c")
    s = lax.axis_index("s")
    E = src_hbm.shape[0]
    N = acc.shape[0]
    K = _SEG_K
    epw = E // _NS
    nch = epw // K
    gsems = (gs0, gs1)
    isems = (is0, is1, is2, is3)
    zv = jnp.zeros((16,), jnp.float32)

    def zb(i, _):
        zbuf[i // 8, pl.ds((i % 8) * 16, 16)] = zv
        return 0

    lax.fori_loop(0, _SEG_ZR * 8, zb, 0)
    rbase = s * _SEG_STRIDE

    for t_local in range(2):
        toff = (c * 2 + t_local) * N
        tvec = jnp.full((16,), toff, jnp.int32)
        for z in range(5):
            pltpu.sync_copy(zbuf, acc.at[pl.ds(rbase + z * _SEG_ZR, _SEG_ZR)])
        plsc.subcore_barrier()

        def issue_idx(ch, q):
            ebase = s * epw + ch * K
            pltpu.async_copy(src_hbm.at[pl.ds(ebase, K)], sidx4.at[q],
                             isems[q])
            pltpu.async_copy(dst_hbm.at[pl.ds(ebase, K)], didx4.at[q],
                             isems[q])

        def wait_idx(ch, q):
            ebase = s * epw + ch * K
            pltpu.make_async_copy(src_hbm.at[pl.ds(ebase, K)], sidx4.at[q],
                                  isems[q]).wait()
            pltpu.make_async_copy(dst_hbm.at[pl.ds(ebase, K)], didx4.at[q],
                                  isems[q]).wait()

        def add_off_and_gather(q, b):
            for j in range(K // 16):
                sidx4[q, pl.ds(j * 16, 16)] = \
                    sidx4[q, pl.ds(j * 16, 16)] + tvec
            pltpu.async_copy(h_hbm.at[sidx4.at[q]], rows2.at[b], gsems[b])

        def wait_gather(q, b):
            pltpu.make_async_copy(
                h_hbm.at[sidx4.at[q]], rows2.at[b], gsems[b]).wait()

        def scatter(q, b):
            pltpu.sync_copy(rows2.at[b], acc.at[didx4.at[q]], add=True)

        for k in range(4):
            issue_idx(k, k)
        for ch in range(2):
            wait_idx(ch, ch)
            add_off_and_gather(ch, ch)

        def quad(i, _):
            for k in range(4):
                q = k
                b = k % 2
                qn = (k + 2) % 4
                ch = 4 * i + k
                wait_gather(q, b)
                scatter(q, b)
                wait_idx(ch + 2, qn)
                add_off_and_gather(qn, b)
                issue_idx(ch + 4, q)
            return 0

        lax.fori_loop(0, nch // 4 - 2, quad, 0)
        for x in range(10):
            ch = nch - 10 + x
            q = ch % 4
            b = ch % 2
            qn = (ch + 2) % 4
            wait_gather(q, b)
            scatter(q, b)
            if x < 8:
                wait_idx(ch + 2, qn)
                add_off_and_gather(qn, b)
            if x < 6:
                issue_idx(ch + 4, q)
        plsc.subcore_barrier()
        for z in range(5):
            rb = rbase + z * _SEG_ZR
            pltpu.sync_copy(acc.at[pl.ds(rb, _SEG_ZR)],
                            out_hbm.at[pl.ds(toff + rb, _SEG_ZR)])
        plsc.subcore_barrier()


def _segsum(h2, src, dst, n):
    tn, d = h2.shape
    return pl.kernel(
        _segsum_body,
        out_type=jax.ShapeDtypeStruct((tn, d), jnp.float32),
        mesh=plsc.VectorSubcoreMesh(core_axis_name="c", subcore_axis_name="s",
                                    num_cores=_NC, num_subcores=_NS),
        scratch_types=[
            pltpu.VMEM((4, _SEG_K), jnp.int32),
            pltpu.VMEM((4, _SEG_K), jnp.int32),
            pltpu.VMEM((2, _SEG_K, d), jnp.float32),
            pltpu.VMEM((_SEG_ZR, d), jnp.float32),
            pltpu.VMEM_SHARED((n, d), jnp.float32),
            pltpu.SemaphoreType.DMA,
            pltpu.SemaphoreType.DMA,
            pltpu.SemaphoreType.DMA,
            pltpu.SemaphoreType.DMA,
            pltpu.SemaphoreType.DMA,
            pltpu.SemaphoreType.DMA,
        ],
    )(h2, src, dst)


_BLK = 2000


def _c0_body(x_ref, degs_ref, h_ref, nin0_ref, nout1_ref, nin1_ref):
    T = x_ref.shape[0]
    deg_out = degs_ref[0, :, 0:1]
    deg_in = degs_ref[1, :, 0:1]
    nout0 = lax.rsqrt(jnp.maximum(deg_out, 1.0))
    nin0_ref[...] = lax.rsqrt(jnp.maximum(deg_in, 1.0))
    nout1_ref[...] = lax.rsqrt(jnp.maximum(deg_out + 1.0, 1.0))
    nin1_ref[...] = lax.rsqrt(jnp.maximum(deg_in + 1.0, 1.0))
    for t in range(T):
        h_ref[t] = x_ref[t] * nout0


def _c0(features, degs):
    t, n, d = features.shape
    nb = n // _BLK
    return pl.pallas_call(
        _c0_body,
        grid=(nb,),
        in_specs=[
            pl.BlockSpec((t, _BLK, d), lambda i: (0, i, 0)),
            pl.BlockSpec((2, _BLK, 128), lambda i: (0, i, 0)),
        ],
        out_specs=[
            pl.BlockSpec((t, _BLK, d), lambda i: (0, i, 0)),
            pl.BlockSpec((_BLK, 1), lambda i: (i, 0)),
            pl.BlockSpec((_BLK, 1), lambda i: (i, 0)),
            pl.BlockSpec((_BLK, 1), lambda i: (i, 0)),
        ],
        out_shape=[
            jax.ShapeDtypeStruct((t, n, d), jnp.float32),
            jax.ShapeDtypeStruct((n, 1), jnp.float32),
            jax.ShapeDtypeStruct((n, 1), jnp.float32),
            jax.ShapeDtypeStruct((n, 1), jnp.float32),
        ],
    )(features, degs)


def _c1_body(agg_ref, nin_ref, nout_ref, w_ref, b_ref, h_ref):
    a = agg_ref[0] * nin_ref[...]
    x1 = jnp.dot(a, w_ref[...], preferred_element_type=jnp.float32) + b_ref[...]
    h_ref[0] = x1 * nout_ref[...]


def _c1(agg0, nin0, nout1, w, b):
    t, n, d = agg0.shape
    h = w.shape[1]
    nb = n // _BLK
    return pl.pallas_call(
        _c1_body,
        grid=(t, nb),
        in_specs=[
            pl.BlockSpec((1, _BLK, d), lambda ti, i: (ti, i, 0)),
            pl.BlockSpec((_BLK, 1), lambda ti, i: (i, 0)),
            pl.BlockSpec((_BLK, 1), lambda ti, i: (i, 0)),
            pl.BlockSpec((d, h), lambda ti, i: (0, 0)),
            pl.BlockSpec((1, h), lambda ti, i: (0, 0)),
        ],
        out_specs=pl.BlockSpec((1, _BLK, h), lambda ti, i: (ti, i, 0)),
        out_shape=jax.ShapeDtypeStruct((t, n, h), jnp.float32),
    )(agg0, nin0, nout1, w, b)


def _c2_body(agg_ref, h1_ref, nin_ref, w_ref, b_ref, out_ref):
    T = agg_ref.shape[0]
    nin = nin_ref[...]
    w = w_ref[...]
    b = b_ref[...]
    for t in range(T):
        a = (agg_ref[t] + h1_ref[t]) * nin
        out_ref[:, t, :] = jnp.dot(
            a, w, preferred_element_type=jnp.float32) + b


def _c2(agg1, h1, nin1, w, b):
    t, n, d = agg1.shape
    h = w.shape[1]
    nb = n // _BLK
    return pl.pallas_call(
        _c2_body,
        grid=(nb,),
        in_specs=[
            pl.BlockSpec((t, _BLK, d), lambda i: (0, i, 0)),
            pl.BlockSpec((t, _BLK, d), lambda i: (0, i, 0)),
            pl.BlockSpec((_BLK, 1), lambda i: (i, 0)),
            pl.BlockSpec((d, h), lambda i: (0, 0)),
            pl.BlockSpec((1, h), lambda i: (0, 0)),
        ],
        out_specs=pl.BlockSpec((_BLK, t, h), lambda i: (i, 0, 0)),
        out_shape=jax.ShapeDtypeStruct((n, t, h), jnp.float32),
    )(agg1, h1, nin1, w, b)


_CHUNK = 500
_DN = (((1,), (1,)), ((), ()))
_PREC = lax.Precision.DEFAULT


def _gru_body(x_ref, wi0, whc, wi1, bi0, bhc, bi1,
              out_ref, gi_ref, hs_ref):
    chunk, T, D = x_ref.shape
    H = D
    c = pl.program_id(0)
    nc = pl.num_programs(0)

    @pl.when(c == 0)
    def _():
        hs_ref[...] = jnp.zeros_like(hs_ref)

    x = x_ref[...].reshape(chunk * T, D)
    gi_ref[...] = lax.dot_general(
        x, wi0[...], _DN, preferred_element_type=jnp.float32,
        precision=lax.Precision.DEFAULT) + bi0[...]

    def step2(i, _):
        base = pl.multiple_of(8 * i, 8)
        g8 = gi_ref[pl.ds(base, 8), :]
        for k in range(2):
            g0 = g8[4 * k:4 * k + 4, :]
            hs = hs_ref[...]
            h0 = hs[:, 0:H]
            h1 = hs[:, H:2 * H]
            gh = jnp.dot(hs, whc[...], preferred_element_type=jnp.float32,
                         precision=_PREC) + bhc[...]
            r0 = jax.nn.sigmoid(g0[:, 0:H] + gh[:, 0:H])
            z0 = jax.nn.sigmoid(g0[:, H:2 * H] + gh[:, H:2 * H])
            n0 = jnp.tanh(g0[:, 2 * H:3 * H] + r0 * gh[:, 2 * H:3 * H])
            h0n = n0 + z0 * (h0 - n0)
            gi1 = lax.dot_general(
                h0n, wi1[...], _DN, preferred_element_type=jnp.float32,
                precision=lax.Precision.DEFAULT) + bi1[...]
            r1 = jax.nn.sigmoid(gi1[:, 0:H] + gh[:, 3 * H:4 * H])
            z1 = jax.nn.sigmoid(gi1[:, H:2 * H] + gh[:, 4 * H:5 * H])
            n1 = jnp.tanh(gi1[:, 2 * H:3 * H] + r1 * gh[:, 5 * H:6 * H])
            h1n = n1 + z1 * (h1 - n1)
            hs_ref[:, 0:H] = h0n
            hs_ref[:, H:2 * H] = h1n
        return 0

    lax.fori_loop(0, chunk // 2, step2, 0)

    @pl.when(c == nc - 1)
    def _():
        out_ref[...] = jnp.maximum(hs_ref[:, H:2 * H], 0.0)


def _gru(x, wi0, whc, wi1, bi0, bhc, bi1):
    n, t, d = x.shape
    h3 = wi0.shape[0]
    nc = n // _CHUNK
    return pl.pallas_call(
        _gru_body,
        grid=(nc,),
        in_specs=[
            pl.BlockSpec((_CHUNK, t, d), lambda c: (c, 0, 0)),
            pl.BlockSpec(wi0.shape, lambda c: (0, 0)),
            pl.BlockSpec(whc.shape, lambda c: (0, 0)),
            pl.BlockSpec(wi1.shape, lambda c: (0, 0)),
            pl.BlockSpec(bi0.shape, lambda c: (0, 0)),
            pl.BlockSpec(bhc.shape, lambda c: (0, 0)),
            pl.BlockSpec(bi1.shape, lambda c: (0, 0)),
        ],
        out_specs=pl.BlockSpec((t, d), lambda c: (0, 0)),
        out_shape=jax.ShapeDtypeStruct((t, d), jnp.float32),
        scratch_shapes=[
            pltpu.VMEM((_CHUNK * t, h3), jnp.float32),
            pltpu.VMEM((t, 2 * d), jnp.float32),
        ],
    )(x, wi0, whc, wi1, bi0, bhc, bi1)


def kernel(features, edge_index, W_conv, b_conv, w_ih0, w_hh0, b_ih0, b_hh0,
           w_ih1, w_hh1, b_ih1, b_hh1):
    t, n, d = features.shape
    h = W_conv.shape[1]
    src = edge_index[0]
    dst = edge_index[1]
    degs = _degrees(src, dst, n)
    h0, nin0, nout1, nin1 = _c0(features, degs)
    agg0 = _segsum(h0.reshape(t * n, d), src, dst, n)
    h1 = _c1(agg0.reshape(t, n, d), nin0, nout1, W_conv, b_conv.reshape(1, h))
    agg1 = _segsum(h1.reshape(t * n, h), src, dst, n)
    x2 = _c2(agg1.reshape(t, n, h), h1, nin1, W_conv, b_conv.reshape(1, h))
    whc = jnp.zeros((2 * h, 6 * h), jnp.float32)
    whc = whc.at[0:h, 0:3 * h].set(w_hh0.T)
    whc = whc.at[h:2 * h, 3 * h:6 * h].set(w_hh1.T)
    bhc = jnp.concatenate([b_hh0, b_hh1]).reshape(1, -1)
    out = _gru(x2, w_ih0, whc, w_ih1,
               b_ih0.reshape(1, -1), bhc, b_ih1.reshape(1, -1))
    return out[:, None, :]
